# baseline (speedup 1.0000x reference)
import jax
import jax.numpy as jnp
from jax import lax
from jax.experimental import pallas as pl
from jax.experimental.pallas import tpu as pltpu
from jax.experimental.pallas import tpu_sc as plsc

N_USERS = 50000
N_ITEMS = 50000
NR = N_USERS + N_ITEMS
D = 64
GW = 16
NG = D // GW
NNZ = 1600000
BATCH = 1024
HIST = 50
HP = 64

NC = 2
NS = 16

NNZ_PAD = 1638400
EPT = NNZ_PAD // NS
BLKE = 6400
NBLK = EPT // BLKE
CK = 128
NCH = BLKE // CK
ZR = 6256
ZR_LAST = NR - 15 * ZR

_mesh = plsc.VectorSubcoreMesh(core_axis_name="c", subcore_axis_name="s")
_sc_params = pltpu.CompilerParams(use_tc_tiling_on_sc=False)


def _spmm_body(row_h, col_h, val_h, x_h, zeros_h, out_h, acc,
               rb, cb, vb, colb0, colb1, rowb0, rowb1, g0, g1,
               sg0, sg1, ss0, ss1):
    c = lax.axis_index("c")
    s = lax.axis_index("s")
    ebase = s * EPT
    colbs = (colb0, colb1)
    rowbs = (rowb0, rowb1)
    gs = (g0, g1)
    sgs = (sg0, sg1)
    sss = (ss0, ss1)

    def one_pass(p, carry):
        g = 2 * p + c
        goff = g * NR

        @pl.when(s < NS - 1)
        def _():
            pltpu.sync_copy(zeros_h, acc.at[pl.ds(s * ZR, ZR)])

        @pl.when(s == NS - 1)
        def _():
            pltpu.sync_copy(zeros_h.at[pl.ds(0, ZR_LAST)],
                            acc.at[pl.ds(15 * ZR, ZR_LAST)])

        plsc.subcore_barrier()

        def prep(ci, k, first=False):
            if not first:
                pltpu.make_async_copy(gs[k], acc.at[rowbs[k]], sss[k]).wait()
            coff = ci * CK
            for j in range(CK // 16):
                cvec = cb[pl.ds(coff + j * 16, 16)]
                colbs[k][pl.ds(j * 16, 16)] = cvec + goff
                rowbs[k][pl.ds(j * 16, 16)] = rb[pl.ds(coff + j * 16, 16)]
            pltpu.async_copy(x_h.at[colbs[k]], gs[k], sgs[k])

        def proc(ci, k):
            pltpu.make_async_copy(x_h.at[colbs[k]], gs[k], sgs[k]).wait()
            coff = ci * CK
            gk = gs[k]
            for j in range(CK // 16):
                vvj = vb[pl.ds(coff + j * 16, 16)]
                for e2 in range(16):
                    e = j * 16 + e2
                    gk[e, pl.ds(0, 16)] = gk[e, pl.ds(0, 16)] * vvj[e2]
            pltpu.async_copy(gk, acc.at[rowbs[k]], sss[k], add=True)

        def one_block(blk, carry2):
            eoff = ebase + blk * BLKE
            pltpu.sync_copy(row_h.at[pl.ds(eoff, BLKE)], rb)
            pltpu.sync_copy(col_h.at[pl.ds(eoff, BLKE)], cb)
            pltpu.sync_copy(val_h.at[pl.ds(eoff, BLKE)], vb)

            prep(0, 0, first=True)
            prep(1, 1, first=True)

            def pair(q, carry3):
                c0 = q * 2
                proc(c0, 0)

                @pl.when(q < NCH // 2 - 1)
                def _():
                    prep(c0 + 2, 0)

                proc(c0 + 1, 1)

                @pl.when(q < NCH // 2 - 1)
                def _():
                    prep(c0 + 3, 1)

                return carry3

            lax.fori_loop(0, NCH // 2, pair, jnp.int32(0))
            pltpu.make_async_copy(g0, acc.at[rowb0], ss0).wait()
            pltpu.make_async_copy(g1, acc.at[rowb1], ss1).wait()
            return carry2

        lax.fori_loop(0, NBLK, one_block, jnp.int32(0))
        plsc.subcore_barrier()

        @pl.when(s < NS - 1)
        def _():
            pltpu.sync_copy(acc.at[pl.ds(s * ZR, ZR)],
                            out_h.at[pl.ds(goff + s * ZR, ZR)])

        @pl.when(s == NS - 1)
        def _():
            pltpu.sync_copy(acc.at[pl.ds(15 * ZR, ZR_LAST)],
                            out_h.at[pl.ds(goff + 15 * ZR, ZR_LAST)])

        return carry

    lax.fori_loop(0, NG // NC, one_pass, jnp.int32(0))


_spmm = pl.kernel(
    _spmm_body,
    out_type=jax.ShapeDtypeStruct((NG * NR, GW), jnp.float32),
    mesh=_mesh,
    compiler_params=_sc_params,
    scratch_types=[
        pltpu.VMEM_SHARED((NR, GW), jnp.float32),
        pltpu.VMEM((BLKE,), jnp.int32),
        pltpu.VMEM((BLKE,), jnp.int32),
        pltpu.VMEM((BLKE,), jnp.float32),
        pltpu.VMEM((CK,), jnp.int32),
        pltpu.VMEM((CK,), jnp.int32),
        pltpu.VMEM((CK,), jnp.int32),
        pltpu.VMEM((CK,), jnp.int32),
        pltpu.VMEM((CK, GW), jnp.float32),
        pltpu.VMEM((CK, GW), jnp.float32),
        pltpu.SemaphoreType.DMA,
        pltpu.SemaphoreType.DMA,
        pltpu.SemaphoreType.DMA,
        pltpu.SemaphoreType.DMA,
    ],
)

_NB = BATCH // (NC * NS)


def _gather_body(hisp_h, useq_h, nxt_h, item_h, x0_h, x1_h, x2_h, x3_h,
                 hsum_h, u_h, t_h,
                 hidx, hg0, hg1, hs, uidx, nidx, ubidx, tbidx, tmp, usum, tsum,
                 sh0, sh1, su):
    c = lax.axis_index("c")
    s = lax.axis_index("s")
    w = c * NS + s
    bbase = w * _NB

    pltpu.sync_copy(hisp_h.at[pl.ds(bbase, _NB)], hidx)
    pltpu.sync_copy(useq_h.at[pl.ds(bbase, _NB)], uidx)
    pltpu.sync_copy(nxt_h.at[pl.ds(bbase, _NB)], nidx)

    def hissue(b, hgk, shk):
        pltpu.async_copy(item_h.at[hidx.at[b]], hgk, shk)

    def hproc(b, hgk, shk):
        pltpu.make_async_copy(item_h.at[hidx.at[b]], hgk, shk).wait()
        for k2 in range(4):
            acc = jnp.zeros((16,), jnp.float32)
            for r in range(HP):
                acc = acc + hgk[r, pl.ds(k2 * 16, 16)]
            hs[pl.ds(b * D + k2 * 16, 16)] = acc

    hissue(0, hg0, sh0)

    def hpair(q, carry):
        b0 = q * 2
        hissue(b0 + 1, hg1, sh1)
        hproc(b0, hg0, sh0)

        @pl.when(q < _NB // 2 - 1)
        def _():
            hissue(b0 + 2, hg0, sh0)

        hproc(b0 + 1, hg1, sh1)
        return carry

    lax.fori_loop(0, _NB // 2, hpair, jnp.int32(0))
    pltpu.sync_copy(hs, hsum_h.at[pl.ds(bbase * D, _NB * D)])

    xs = (x0_h, x1_h, x2_h, x3_h)

    def gbody(g, carry):
        for h in range(_NB // 16):
            uv = uidx[pl.ds(h * 16, 16)]
            ubidx[pl.ds(h * 16, 16)] = uv + g * NR
            nv = nidx[pl.ds(h * 16, 16)]
            tbidx[pl.ds(h * 16, 16)] = nv + (g * NR + N_USERS - 1)
        for r in range(_NB):
            usum[r, pl.ds(0, 16)] = jnp.zeros((16,), jnp.float32)
            tsum[r, pl.ds(0, 16)] = jnp.zeros((16,), jnp.float32)
        for xl in xs:
            pltpu.async_copy(xl.at[ubidx], tmp, su).wait()
            for r in range(_NB):
                usum[r, pl.ds(0, 16)] = usum[r, pl.ds(0, 16)] + tmp[r, pl.ds(0, 16)]
            pltpu.async_copy(xl.at[tbidx], tmp, su).wait()
            for r in range(_NB):
                tsum[r, pl.ds(0, 16)] = tsum[r, pl.ds(0, 16)] + tmp[r, pl.ds(0, 16)]
        pltpu.sync_copy(usum, u_h.at[pl.ds(g * BATCH + bbase, _NB)])
        pltpu.sync_copy(tsum, t_h.at[pl.ds(g * BATCH + bbase, _NB)])
        return carry

    lax.fori_loop(0, NG, gbody, jnp.int32(0))


_gathers = pl.kernel(
    _gather_body,
    out_type=(
        jax.ShapeDtypeStruct((BATCH * D,), jnp.float32),
        jax.ShapeDtypeStruct((NG * BATCH, GW), jnp.float32),
        jax.ShapeDtypeStruct((NG * BATCH, GW), jnp.float32),
    ),
    mesh=_mesh,
    compiler_params=_sc_params,
    scratch_types=[
        pltpu.VMEM((_NB, HP), jnp.int32),
        pltpu.VMEM((HP, D), jnp.float32),
        pltpu.VMEM((HP, D), jnp.float32),
        pltpu.VMEM((_NB * D,), jnp.float32),
        pltpu.VMEM((_NB,), jnp.int32),
        pltpu.VMEM((_NB,), jnp.int32),
        pltpu.VMEM((_NB,), jnp.int32),
        pltpu.VMEM((_NB,), jnp.int32),
        pltpu.VMEM((_NB, GW), jnp.float32),
        pltpu.VMEM((_NB, GW), jnp.float32),
        pltpu.VMEM((_NB, GW), jnp.float32),
        pltpu.SemaphoreType.DMA,
        pltpu.SemaphoreType.DMA,
        pltpu.SemaphoreType.DMA,
    ],
)

IT = 512
NIT = (N_ITEMS + IT - 1) // IT
NEG = -1e30


def _score_body(us, ts, hsum, hisp, x0, x1, x2, x3, out,
                pbf, tdot, m, sacc):
    i = pl.program_id(0)

    @pl.when(i == 0)
    def _():
        cntf = jnp.sum((hisp[...] != 0).astype(jnp.float32), axis=1,
                       keepdims=True)
        p = us[...] * jnp.float32(1.0 / 16.0) + hsum[...] * (jnp.float32(0.25) / cntf)
        pbf[...] = p.astype(jnp.bfloat16)
        tdot[...] = jnp.sum(p * ts[...], axis=1, keepdims=True)
        m[...] = jnp.full((BATCH, 1), jnp.float32(NEG))
        sacc[...] = jnp.zeros((BATCH, 1), jnp.float32)

    it = (x0[...] + x1[...] + x2[...] + x3[...]).astype(jnp.bfloat16)
    sc = lax.dot_general(pbf[...], it, (((1,), (1,)), ((), ())),
                         preferred_element_type=jnp.float32)
    col = i * IT + lax.broadcasted_iota(jnp.int32, (1, IT), 1)
    sc = jnp.where(col < N_ITEMS, sc, jnp.float32(NEG))
    bm = jnp.max(sc, axis=1, keepdims=True)
    mnew = jnp.maximum(m[...], bm)
    sacc[...] = (sacc[...] * jnp.exp(m[...] - mnew)
                 + jnp.sum(jnp.exp(sc - mnew), axis=1, keepdims=True))
    m[...] = mnew

    @pl.when(i == pl.num_programs(0) - 1)
    def _():
        logz = jnp.log(sacc[...]) + m[...]
        out[...] = jnp.mean(logz - tdot[...]).reshape(1, 1)


_score = pl.pallas_call(
    _score_body,
    grid=(NIT,),
    in_specs=[
        pl.BlockSpec((BATCH, D), lambda i: (0, 0)),
        pl.BlockSpec((BATCH, D), lambda i: (0, 0)),
        pl.BlockSpec((BATCH, D), lambda i: (0, 0)),
        pl.BlockSpec((BATCH, HP), lambda i: (0, 0)),
        pl.BlockSpec((IT, D), lambda i: (i, 0)),
        pl.BlockSpec((IT, D), lambda i: (i, 0)),
        pl.BlockSpec((IT, D), lambda i: (i, 0)),
        pl.BlockSpec((IT, D), lambda i: (i, 0)),
    ],
    out_specs=pl.BlockSpec((1, 1), lambda i: (0, 0)),
    out_shape=jax.ShapeDtypeStruct((1, 1), jnp.float32),
    scratch_shapes=[
        pltpu.VMEM((BATCH, D), jnp.bfloat16),
        pltpu.VMEM((BATCH, 1), jnp.float32),
        pltpu.VMEM((BATCH, 1), jnp.float32),
        pltpu.VMEM((BATCH, 1), jnp.float32),
    ],
)


def _to_grouped(x):
    return x.reshape(NR, NG, GW).transpose(1, 0, 2).reshape(NG * NR, GW)


def kernel(user_seqs, his_seqs, next_items, user_emb_w, item_emb_w,
           adj_row, adj_col, adj_val):
    x0 = jnp.concatenate([user_emb_w, item_emb_w[1:]], axis=0)
    x0a = _to_grouped(x0)
    pad = NNZ_PAD - NNZ
    rowp = jnp.pad(adj_row, (0, pad))
    colp = jnp.pad(adj_col, (0, pad))
    valp = jnp.pad(adj_val, (0, pad))
    zrows = jnp.zeros((ZR, GW), jnp.float32)

    x1a = _spmm(rowp, colp, valp, x0a, zrows)
    x2a = _spmm(rowp, colp, valp, x1a, zrows)
    x3a = _spmm(rowp, colp, valp, x2a, zrows)

    hisp = jnp.pad(his_seqs, ((0, 0), (0, HP - HIST)))
    hsum_f, u_out, t_out = _gathers(hisp, user_seqs, next_items, item_emb_w,
                                    x0a, x1a, x2a, x3a)

    def _ungroup(y):
        return y.reshape(NG, BATCH, GW).transpose(1, 0, 2).reshape(BATCH, D)

    u_sum = _ungroup(u_out)
    t_sum = _ungroup(t_out)

    items = [
        jnp.concatenate(
            [xa[g * NR + N_USERS:(g + 1) * NR] for g in range(NG)], axis=1)
        for xa in (x0a, x1a, x2a, x3a)
    ]

    loss = _score(u_sum, t_sum, hsum_f.reshape(BATCH, D), hisp, *items)
    return loss[0, 0]


# speedup vs baseline: 1.1144x; 1.1144x over previous
import jax
import jax.numpy as jnp
from jax import lax
from jax.experimental import pallas as pl
from jax.experimental.pallas import tpu as pltpu
from jax.experimental.pallas import tpu_sc as plsc

N_USERS = 50000
N_ITEMS = 50000
NR = N_USERS + N_ITEMS
D = 64
GW = 16
NG = D // GW
NNZ = 1600000
BATCH = 1024
HIST = 50
HP = 64

NC = 2
NS = 16

NNZ_PAD = 1638400
EPT = NNZ_PAD // NS
BLKE = 5120
NBLK = EPT // BLKE
CK = 128
NCH = BLKE // CK
NBUF = 4
ZR = 6256
ZR_LAST = NR - 15 * ZR

_mesh = plsc.VectorSubcoreMesh(core_axis_name="c", subcore_axis_name="s")
_sc_params = pltpu.CompilerParams(use_tc_tiling_on_sc=False)


def _spmm_body(row_h, col_h, val_h, x_h, zeros_h, out_h, acc,
               rb, cb, vb, colb0, colb1, colb2, colb3,
               rowb0, rowb1, rowb2, rowb3, g0, g1, g2, g3,
               sg0, sg1, sg2, sg3, ss0, ss1, ss2, ss3):
    c = lax.axis_index("c")
    s = lax.axis_index("s")
    ebase = s * EPT
    colbs = (colb0, colb1, colb2, colb3)
    rowbs = (rowb0, rowb1, rowb2, rowb3)
    gs = (g0, g1, g2, g3)
    sgs = (sg0, sg1, sg2, sg3)
    sss = (ss0, ss1, ss2, ss3)

    def one_pass(p, carry):
        g = 2 * p + c
        goff = g * NR

        @pl.when(s < NS - 1)
        def _():
            pltpu.sync_copy(zeros_h, acc.at[pl.ds(s * ZR, ZR)])

        @pl.when(s == NS - 1)
        def _():
            pltpu.sync_copy(zeros_h.at[pl.ds(0, ZR_LAST)],
                            acc.at[pl.ds(15 * ZR, ZR_LAST)])

        plsc.subcore_barrier()

        def prep(ci, k, first=False):
            if not first:
                pltpu.make_async_copy(gs[k], acc.at[rowbs[k]], sss[k]).wait()
            coff = ci * CK
            for j in range(CK // 16):
                cvec = cb[pl.ds(coff + j * 16, 16)]
                colbs[k][pl.ds(j * 16, 16)] = cvec + goff
                rowbs[k][pl.ds(j * 16, 16)] = rb[pl.ds(coff + j * 16, 16)]
            pltpu.async_copy(x_h.at[colbs[k]], gs[k], sgs[k])

        def proc(ci, k):
            pltpu.make_async_copy(x_h.at[colbs[k]], gs[k], sgs[k]).wait()
            coff = ci * CK
            gk = gs[k]
            for j in range(CK // 16):
                vvj = vb[pl.ds(coff + j * 16, 16)]
                for e2 in range(16):
                    e = j * 16 + e2
                    gk[e, pl.ds(0, 16)] = gk[e, pl.ds(0, 16)] * vvj[e2]
            pltpu.async_copy(gk, acc.at[rowbs[k]], sss[k], add=True)

        def one_block(blk, carry2):
            eoff = ebase + blk * BLKE
            pltpu.sync_copy(row_h.at[pl.ds(eoff, BLKE)], rb)
            pltpu.sync_copy(col_h.at[pl.ds(eoff, BLKE)], cb)
            pltpu.sync_copy(val_h.at[pl.ds(eoff, BLKE)], vb)

            for k in range(NBUF):
                prep(k, k, first=True)

            def quad(q, carry3):
                c0 = q * NBUF
                for k in range(NBUF):
                    proc(c0 + k, k)

                    @pl.when(c0 + k + NBUF < NCH)
                    def _(c0=c0, k=k):
                        prep(c0 + k + NBUF, k)

                return carry3

            lax.fori_loop(0, NCH // NBUF, quad, jnp.int32(0))
            for k in range(NBUF):
                pltpu.make_async_copy(gs[k], acc.at[rowbs[k]], sss[k]).wait()
            return carry2

        lax.fori_loop(0, NBLK, one_block, jnp.int32(0))
        plsc.subcore_barrier()

        @pl.when(s < NS - 1)
        def _():
            pltpu.sync_copy(acc.at[pl.ds(s * ZR, ZR)],
                            out_h.at[pl.ds(goff + s * ZR, ZR)])

        @pl.when(s == NS - 1)
        def _():
            pltpu.sync_copy(acc.at[pl.ds(15 * ZR, ZR_LAST)],
                            out_h.at[pl.ds(goff + 15 * ZR, ZR_LAST)])

        return carry

    lax.fori_loop(0, NG // NC, one_pass, jnp.int32(0))


_spmm = pl.kernel(
    _spmm_body,
    out_type=jax.ShapeDtypeStruct((NG * NR, GW), jnp.float32),
    mesh=_mesh,
    compiler_params=_sc_params,
    scratch_types=[
        pltpu.VMEM_SHARED((NR, GW), jnp.float32),
        pltpu.VMEM((BLKE,), jnp.int32),
        pltpu.VMEM((BLKE,), jnp.int32),
        pltpu.VMEM((BLKE,), jnp.float32),
        pltpu.VMEM((CK,), jnp.int32),
        pltpu.VMEM((CK,), jnp.int32),
        pltpu.VMEM((CK,), jnp.int32),
        pltpu.VMEM((CK,), jnp.int32),
        pltpu.VMEM((CK,), jnp.int32),
        pltpu.VMEM((CK,), jnp.int32),
        pltpu.VMEM((CK,), jnp.int32),
        pltpu.VMEM((CK,), jnp.int32),
        pltpu.VMEM((CK, GW), jnp.float32),
        pltpu.VMEM((CK, GW), jnp.float32),
        pltpu.VMEM((CK, GW), jnp.float32),
        pltpu.VMEM((CK, GW), jnp.float32),
        pltpu.SemaphoreType.DMA,
        pltpu.SemaphoreType.DMA,
        pltpu.SemaphoreType.DMA,
        pltpu.SemaphoreType.DMA,
        pltpu.SemaphoreType.DMA,
        pltpu.SemaphoreType.DMA,
        pltpu.SemaphoreType.DMA,
        pltpu.SemaphoreType.DMA,
    ],
)

_NB = BATCH // (NC * NS)


def _gather_body(hisp_h, useq_h, nxt_h, item_h, x0_h, x1_h, x2_h, x3_h,
                 hsum_h, u_h, t_h,
                 hidx, hg0, hg1, hs, uidx, nidx, ubidx, tbidx, tmp, usum, tsum,
                 sh0, sh1, su):
    c = lax.axis_index("c")
    s = lax.axis_index("s")
    w = c * NS + s
    bbase = w * _NB

    pltpu.sync_copy(hisp_h.at[pl.ds(bbase, _NB)], hidx)
    pltpu.sync_copy(useq_h.at[pl.ds(bbase, _NB)], uidx)
    pltpu.sync_copy(nxt_h.at[pl.ds(bbase, _NB)], nidx)

    def hissue(b, hgk, shk):
        pltpu.async_copy(item_h.at[hidx.at[b]], hgk, shk)

    def hproc(b, hgk, shk):
        pltpu.make_async_copy(item_h.at[hidx.at[b]], hgk, shk).wait()
        for k2 in range(4):
            acc = jnp.zeros((16,), jnp.float32)
            for r in range(HP):
                acc = acc + hgk[r, pl.ds(k2 * 16, 16)]
            hs[pl.ds(b * D + k2 * 16, 16)] = acc

    hissue(0, hg0, sh0)

    def hpair(q, carry):
        b0 = q * 2
        hissue(b0 + 1, hg1, sh1)
        hproc(b0, hg0, sh0)

        @pl.when(q < _NB // 2 - 1)
        def _():
            hissue(b0 + 2, hg0, sh0)

        hproc(b0 + 1, hg1, sh1)
        return carry

    lax.fori_loop(0, _NB // 2, hpair, jnp.int32(0))
    pltpu.sync_copy(hs, hsum_h.at[pl.ds(bbase * D, _NB * D)])

    xs = (x0_h, x1_h, x2_h, x3_h)

    def gbody(g, carry):
        for h in range(_NB // 16):
            uv = uidx[pl.ds(h * 16, 16)]
            ubidx[pl.ds(h * 16, 16)] = uv + g * NR
            nv = nidx[pl.ds(h * 16, 16)]
            tbidx[pl.ds(h * 16, 16)] = nv + (g * NR + N_USERS - 1)
        for r in range(_NB):
            usum[r, pl.ds(0, 16)] = jnp.zeros((16,), jnp.float32)
            tsum[r, pl.ds(0, 16)] = jnp.zeros((16,), jnp.float32)
        for xl in xs:
            pltpu.async_copy(xl.at[ubidx], tmp, su).wait()
            for r in range(_NB):
                usum[r, pl.ds(0, 16)] = usum[r, pl.ds(0, 16)] + tmp[r, pl.ds(0, 16)]
            pltpu.async_copy(xl.at[tbidx], tmp, su).wait()
            for r in range(_NB):
                tsum[r, pl.ds(0, 16)] = tsum[r, pl.ds(0, 16)] + tmp[r, pl.ds(0, 16)]
        pltpu.sync_copy(usum, u_h.at[pl.ds(g * BATCH + bbase, _NB)])
        pltpu.sync_copy(tsum, t_h.at[pl.ds(g * BATCH + bbase, _NB)])
        return carry

    lax.fori_loop(0, NG, gbody, jnp.int32(0))


_gathers = pl.kernel(
    _gather_body,
    out_type=(
        jax.ShapeDtypeStruct((BATCH * D,), jnp.float32),
        jax.ShapeDtypeStruct((NG * BATCH, GW), jnp.float32),
        jax.ShapeDtypeStruct((NG * BATCH, GW), jnp.float32),
    ),
    mesh=_mesh,
    compiler_params=_sc_params,
    scratch_types=[
        pltpu.VMEM((_NB, HP), jnp.int32),
        pltpu.VMEM((HP, D), jnp.float32),
        pltpu.VMEM((HP, D), jnp.float32),
        pltpu.VMEM((_NB * D,), jnp.float32),
        pltpu.VMEM((_NB,), jnp.int32),
        pltpu.VMEM((_NB,), jnp.int32),
        pltpu.VMEM((_NB,), jnp.int32),
        pltpu.VMEM((_NB,), jnp.int32),
        pltpu.VMEM((_NB, GW), jnp.float32),
        pltpu.VMEM((_NB, GW), jnp.float32),
        pltpu.VMEM((_NB, GW), jnp.float32),
        pltpu.SemaphoreType.DMA,
        pltpu.SemaphoreType.DMA,
        pltpu.SemaphoreType.DMA,
    ],
)

IT = 512
NIT = (N_ITEMS + IT - 1) // IT
NEG = -1e30


def _score_body(us, ts, hsum, hisp, x0, x1, x2, x3, out,
                pbf, tdot, m, sacc):
    i = pl.program_id(0)

    @pl.when(i == 0)
    def _():
        cntf = jnp.sum((hisp[...] != 0).astype(jnp.float32), axis=1,
                       keepdims=True)
        p = us[...] * jnp.float32(1.0 / 16.0) + hsum[...] * (jnp.float32(0.25) / cntf)
        pbf[...] = p.astype(jnp.bfloat16)
        tdot[...] = jnp.sum(p * ts[...], axis=1, keepdims=True)
        m[...] = jnp.full((BATCH, 1), jnp.float32(NEG))
        sacc[...] = jnp.zeros((BATCH, 1), jnp.float32)

    it = (x0[...] + x1[...] + x2[...] + x3[...]).astype(jnp.bfloat16)
    sc = lax.dot_general(pbf[...], it, (((1,), (1,)), ((), ())),
                         preferred_element_type=jnp.float32)
    col = i * IT + lax.broadcasted_iota(jnp.int32, (1, IT), 1)
    sc = jnp.where(col < N_ITEMS, sc, jnp.float32(NEG))
    bm = jnp.max(sc, axis=1, keepdims=True)
    mnew = jnp.maximum(m[...], bm)
    sacc[...] = (sacc[...] * jnp.exp(m[...] - mnew)
                 + jnp.sum(jnp.exp(sc - mnew), axis=1, keepdims=True))
    m[...] = mnew

    @pl.when(i == pl.num_programs(0) - 1)
    def _():
        logz = jnp.log(sacc[...]) + m[...]
        out[...] = jnp.mean(logz - tdot[...]).reshape(1, 1)


_score = pl.pallas_call(
    _score_body,
    grid=(NIT,),
    in_specs=[
        pl.BlockSpec((BATCH, D), lambda i: (0, 0)),
        pl.BlockSpec((BATCH, D), lambda i: (0, 0)),
        pl.BlockSpec((BATCH, D), lambda i: (0, 0)),
        pl.BlockSpec((BATCH, HP), lambda i: (0, 0)),
        pl.BlockSpec((IT, D), lambda i: (i, 0)),
        pl.BlockSpec((IT, D), lambda i: (i, 0)),
        pl.BlockSpec((IT, D), lambda i: (i, 0)),
        pl.BlockSpec((IT, D), lambda i: (i, 0)),
    ],
    out_specs=pl.BlockSpec((1, 1), lambda i: (0, 0)),
    out_shape=jax.ShapeDtypeStruct((1, 1), jnp.float32),
    scratch_shapes=[
        pltpu.VMEM((BATCH, D), jnp.bfloat16),
        pltpu.VMEM((BATCH, 1), jnp.float32),
        pltpu.VMEM((BATCH, 1), jnp.float32),
        pltpu.VMEM((BATCH, 1), jnp.float32),
    ],
)


def _to_grouped(x):
    return x.reshape(NR, NG, GW).transpose(1, 0, 2).reshape(NG * NR, GW)


def kernel(user_seqs, his_seqs, next_items, user_emb_w, item_emb_w,
           adj_row, adj_col, adj_val):
    x0 = jnp.concatenate([user_emb_w, item_emb_w[1:]], axis=0)
    x0a = _to_grouped(x0)
    pad = NNZ_PAD - NNZ
    rowp = jnp.pad(adj_row, (0, pad))
    colp = jnp.pad(adj_col, (0, pad))
    valp = jnp.pad(adj_val, (0, pad))
    zrows = jnp.zeros((ZR, GW), jnp.float32)

    x1a = _spmm(rowp, colp, valp, x0a, zrows)
    x2a = _spmm(rowp, colp, valp, x1a, zrows)
    x3a = _spmm(rowp, colp, valp, x2a, zrows)

    hisp = jnp.pad(his_seqs, ((0, 0), (0, HP - HIST)))
    hsum_f, u_out, t_out = _gathers(hisp, user_seqs, next_items, item_emb_w,
                                    x0a, x1a, x2a, x3a)

    def _ungroup(y):
        return y.reshape(NG, BATCH, GW).transpose(1, 0, 2).reshape(BATCH, D)

    u_sum = _ungroup(u_out)
    t_sum = _ungroup(t_out)

    items = [
        jnp.concatenate(
            [xa[g * NR + N_USERS:(g + 1) * NR] for g in range(NG)], axis=1)
        for xa in (x0a, x1a, x2a, x3a)
    ]

    loss = _score(u_sum, t_sum, hsum_f.reshape(BATCH, D), hisp, *items)
    return loss[0, 0]
